# baseline (speedup 1.0000x reference)
import functools

import jax
import jax.numpy as jnp
from jax import lax
from jax.experimental import pallas as pl
from jax.experimental.pallas import tpu as pltpu
from jax.experimental.pallas import tpu_sc as plsc

N = 2048
C = 256
E = 32
FF = 1024
BLK = 128
NCHUNK = N // BLK
NBLK = 63
P_PAD = NBLK * BLK
NA = 2 * N


def _route_body(x_ref, wg_ref, rs_ref, pall_ref, w0_ref, w1_ref,
                eoff_ref, eblk_ref):
    x = x_ref[...]
    wg = wg_ref[...]
    logits = lax.dot_general(x, wg, (((1,), (1,)), ((), ())),
                             preferred_element_type=jnp.float32,
                             precision=lax.Precision.DEFAULT)

    lane = lax.broadcasted_iota(jnp.int32, (N, E), 1)
    m1 = jnp.max(logits, axis=1, keepdims=True)
    i1 = jnp.min(jnp.where(logits == m1, lane, E), axis=1, keepdims=True)
    masked = jnp.where(lane == i1, -jnp.inf, logits)
    m2 = jnp.max(masked, axis=1, keepdims=True)
    i2 = jnp.min(jnp.where(masked == m2, lane, E), axis=1, keepdims=True)

    e2 = jnp.exp(m2 - m1)
    w0 = 1.0 / (1.0 + e2)
    w1 = e2 / (1.0 + e2)
    w0_ref[...] = w0
    w1_ref[...] = w1
    rs_ref[...] = (jnp.where(lane == i1, w0, 0.0)
                   + jnp.where(lane == i2, w1, 0.0))

    oh0 = (lane == i1).astype(jnp.float32)
    oh1 = (lane == i2).astype(jnp.float32)
    oh0c = oh0.reshape(NCHUNK, BLK, E)
    oh1c = oh1.reshape(NCHUNK, BLK, E)
    s0 = jnp.sum(oh0c, axis=1)
    s1 = jnp.sum(oh1c, axis=1)
    tot0 = jnp.sum(s0, axis=0, keepdims=True)
    counts = tot0 + jnp.sum(s1, axis=0, keepdims=True)

    nblk = jnp.floor((counts + (BLK - 1)) * (1.0 / BLK))
    er = lax.broadcasted_iota(jnp.int32, (E, E), 0)
    ec = lax.broadcasted_iota(jnp.int32, (E, E), 1)
    upper = (er < ec).astype(jnp.float32)
    blkoff = lax.dot_general(nblk, upper, (((1,), (0,)), ((), ())),
                             preferred_element_type=jnp.float32)
    aoff = blkoff * float(BLK)
    eoff_ref[...] = blkoff.astype(jnp.int32)
    eblk_ref[...] = nblk.astype(jnp.int32)

    cr = lax.broadcasted_iota(jnp.int32, (NCHUNK, NCHUNK), 0)
    cc = lax.broadcasted_iota(jnp.int32, (NCHUNK, NCHUNK), 1)
    lc = (cc < cr).astype(jnp.float32)
    c0 = jnp.dot(lc, s0, preferred_element_type=jnp.float32)
    c1 = jnp.dot(lc, s1, preferred_element_type=jnp.float32) + tot0

    rr = lax.broadcasted_iota(jnp.int32, (BLK, BLK), 0)
    rc = lax.broadcasted_iota(jnp.int32, (BLK, BLK), 1)
    ls = (rc < rr).astype(jnp.float32)

    for c in range(NCHUNK):
        ex0 = jnp.dot(ls, oh0c[c], preferred_element_type=jnp.float32)
        ex1 = jnp.dot(ls, oh1c[c], preferred_element_type=jnp.float32)
        pos0 = jnp.sum(oh0c[c] * (ex0 + c0[c:c + 1, :] + aoff),
                       axis=1, keepdims=True)
        pos1 = jnp.sum(oh1c[c] * (ex1 + c1[c:c + 1, :] + aoff),
                       axis=1, keepdims=True)
        pall_ref[c * BLK:(c + 1) * BLK, :] = pos0.astype(jnp.int32)
        pall_ref[N + c * BLK:N + (c + 1) * BLK, :] = pos1.astype(jnp.int32)


def _route(x_flat, wg):
    outs = (
        jax.ShapeDtypeStruct((N, E), jnp.float32),
        jax.ShapeDtypeStruct((NA, 1), jnp.int32),
        jax.ShapeDtypeStruct((N, 1), jnp.float32),
        jax.ShapeDtypeStruct((N, 1), jnp.float32),
        jax.ShapeDtypeStruct((1, E), jnp.int32),
        jax.ShapeDtypeStruct((1, E), jnp.int32),
    )
    return pl.pallas_call(_route_body, out_shape=outs)(x_flat, wg)


_NW = 32
_TOK_W = N // _NW
_ROWS_W = NA // _NW


def _sc_mesh():
    return plsc.VectorSubcoreMesh(core_axis_name="c", subcore_axis_name="s")


def _sc_scatter(x_flat, pall):
    @functools.partial(
        pl.kernel,
        mesh=_sc_mesh(),
        out_type=jax.ShapeDtypeStruct((P_PAD, C), jnp.float32),
        scratch_types=[
            pltpu.VMEM((_ROWS_W,), jnp.int32),
            pltpu.VMEM((_ROWS_W, C), jnp.float32),
            pltpu.SemaphoreType.DMA,
            pltpu.SemaphoreType.DMA,
        ],
    )
    def k(x_hbm, idx_hbm, out_hbm, idx_v, rows_v, sem_i, sem_x):
        wid = lax.axis_index("s") * 2 + lax.axis_index("c")
        kk = wid % 2
        ww = wid // 2
        ci = pltpu.async_copy(idx_hbm.at[pl.ds(kk * N + ww * _ROWS_W, _ROWS_W)],
                              idx_v, sem_i)
        cx = pltpu.async_copy(x_hbm.at[pl.ds(ww * _ROWS_W, _ROWS_W)],
                              rows_v, sem_x)
        ci.wait()
        cx.wait()
        pltpu.sync_copy(rows_v, out_hbm.at[idx_v])

    return k(x_flat, pall)


def _sc_gather(table, pall):
    @functools.partial(
        pl.kernel,
        mesh=_sc_mesh(),
        out_type=jax.ShapeDtypeStruct((NA, C), jnp.float32),
        scratch_types=[
            pltpu.VMEM((_ROWS_W,), jnp.int32),
            pltpu.VMEM((_ROWS_W, C), jnp.float32),
            pltpu.SemaphoreType.DMA,
        ],
    )
    def k(tab_hbm, idx_hbm, out_hbm, idx_v, rows_v, sem):
        wid = lax.axis_index("s") * 2 + lax.axis_index("c")
        base = wid * _ROWS_W
        pltpu.sync_copy(idx_hbm.at[pl.ds(base, _ROWS_W)], idx_v)
        pltpu.async_copy(tab_hbm.at[idx_v], rows_v, sem).wait()
        pltpu.sync_copy(rows_v, out_hbm.at[pl.ds(base, _ROWS_W)])

    return k(table, pall)


EPS = 8


def _gmm_step(eoff_ref, eblk_ref, xs_ref, w1_ref, w2_ref, out_ref):
    s = pl.program_id(0)

    def _mlp(w1, w2, r0, m):
        xb = xs_ref[pl.ds(r0, m), :]
        h = lax.dot_general(xb, w1, (((1,), (1,)), ((), ())),
                            preferred_element_type=jnp.float32,
                            precision=lax.Precision.DEFAULT)
        h = jnp.square(jnp.maximum(h, 0.0))
        out_ref[pl.ds(r0, m), :] = lax.dot_general(
            h, w2, (((1,), (1,)), ((), ())),
            preferred_element_type=jnp.float32,
            precision=lax.Precision.DEFAULT)

    for i in range(EPS):
        e = s * EPS + i
        off = eoff_ref[e]
        nb = eblk_ref[e]
        w1 = w1_ref[i]
        w2 = w2_ref[i]

        def body(j, carry, w1=w1, w2=w2, off=off):
            _mlp(w1, w2, off * BLK + j * (2 * BLK), 2 * BLK)
            return carry

        lax.fori_loop(0, nb // 2, body, 0)

        @pl.when(nb % 2 == 1)
        def _tail(w1=w1, w2=w2, off=off, nb=nb):
            _mlp(w1, w2, (off + nb - 1) * BLK, BLK)


def _gmm(eoff, eblk, xs, w1, w2):
    grid_spec = pltpu.PrefetchScalarGridSpec(
        num_scalar_prefetch=2,
        grid=(E // EPS,),
        in_specs=[
            pl.BlockSpec((P_PAD, C), lambda s, o, nb: (0, 0)),
            pl.BlockSpec((EPS, FF, C), lambda s, o, nb: (s, 0, 0)),
            pl.BlockSpec((EPS, C, FF), lambda s, o, nb: (s, 0, 0)),
        ],
        out_specs=pl.BlockSpec((P_PAD, C), lambda s, o, nb: (0, 0)),
    )
    return pl.pallas_call(
        _gmm_step,
        grid_spec=grid_spec,
        out_shape=jax.ShapeDtypeStruct((P_PAD, C), jnp.float32),
    )(eoff, eblk, xs, w1, w2)


def _combine_body(g_ref, w0_ref, w1_ref, out_ref):
    out_ref[...] = (g_ref[0:N, :] * w0_ref[...]
                    + g_ref[N:NA, :] * w1_ref[...])


def _combine(g, w0, w1):
    return pl.pallas_call(
        _combine_body,
        out_shape=jax.ShapeDtypeStruct((N, C), jnp.float32),
    )(g, w0, w1)


def kernel(x, Wg, W1, W2):
    bsz, t, c = x.shape
    x_flat = x.reshape(N, C)
    rs, pall, w0, w1, eoff, eblk = _route(x_flat, Wg)
    pf = pall.reshape(-1)
    xs = _sc_scatter(x_flat, pf)
    outs = _gmm(eoff.reshape(-1), eblk.reshape(-1), xs, W1, W2)
    g = _sc_gather(outs, pf)
    out = _combine(g, w0, w1)
    return out.reshape(bsz, t, c), rs


# speedup vs baseline: 1.0587x; 1.0587x over previous
import functools

import jax
import jax.numpy as jnp
from jax import lax
from jax.experimental import pallas as pl
from jax.experimental.pallas import tpu as pltpu
from jax.experimental.pallas import tpu_sc as plsc

N = 2048
C = 256
E = 32
FF = 1024
BLK = 128
NCHUNK = N // BLK
NBLK = 63
P_PAD = NBLK * BLK
NA = 2 * N


def _route_body(x_ref, wg_ref, rs_ref, pall_ref, w0_ref, w1_ref,
                eoff_ref, eblk_ref):
    x = x_ref[...]
    wg = wg_ref[...]
    logits = lax.dot_general(x, wg, (((1,), (1,)), ((), ())),
                             preferred_element_type=jnp.float32,
                             precision=lax.Precision.DEFAULT)

    lane = lax.broadcasted_iota(jnp.int32, (N, E), 1)
    m1 = jnp.max(logits, axis=1, keepdims=True)
    i1 = jnp.min(jnp.where(logits == m1, lane, E), axis=1, keepdims=True)
    masked = jnp.where(lane == i1, -jnp.inf, logits)
    m2 = jnp.max(masked, axis=1, keepdims=True)
    i2 = jnp.min(jnp.where(masked == m2, lane, E), axis=1, keepdims=True)

    e2 = jnp.exp(m2 - m1)
    w0 = 1.0 / (1.0 + e2)
    w1 = e2 / (1.0 + e2)
    w0_ref[...] = w0
    w1_ref[...] = w1
    rs_ref[...] = (jnp.where(lane == i1, w0, 0.0)
                   + jnp.where(lane == i2, w1, 0.0))

    oh0 = (lane == i1).astype(jnp.float32)
    oh1 = (lane == i2).astype(jnp.float32)
    oh0c = oh0.reshape(NCHUNK, BLK, E)
    oh1c = oh1.reshape(NCHUNK, BLK, E)
    s0 = jnp.sum(oh0c, axis=1)
    s1 = jnp.sum(oh1c, axis=1)
    tot0 = jnp.sum(s0, axis=0, keepdims=True)
    counts = tot0 + jnp.sum(s1, axis=0, keepdims=True)

    nblk = jnp.floor((counts + (BLK - 1)) * (1.0 / BLK))
    er = lax.broadcasted_iota(jnp.int32, (E, E), 0)
    ec = lax.broadcasted_iota(jnp.int32, (E, E), 1)
    upper = (er < ec).astype(jnp.float32)
    blkoff = lax.dot_general(nblk, upper, (((1,), (0,)), ((), ())),
                             preferred_element_type=jnp.float32)
    aoff = blkoff * float(BLK)
    eoff_ref[...] = blkoff.astype(jnp.int32)
    eblk_ref[...] = nblk.astype(jnp.int32)

    cr = lax.broadcasted_iota(jnp.int32, (NCHUNK, NCHUNK), 0)
    cc = lax.broadcasted_iota(jnp.int32, (NCHUNK, NCHUNK), 1)
    lc = (cc < cr).astype(jnp.float32)
    c0 = jnp.dot(lc, s0, preferred_element_type=jnp.float32)
    c1 = jnp.dot(lc, s1, preferred_element_type=jnp.float32) + tot0

    rr = lax.broadcasted_iota(jnp.int32, (BLK, BLK), 0)
    rc = lax.broadcasted_iota(jnp.int32, (BLK, BLK), 1)
    ls = (rc < rr).astype(jnp.float32)

    for c in range(NCHUNK):
        ex0 = jnp.dot(ls, oh0c[c], preferred_element_type=jnp.float32)
        ex1 = jnp.dot(ls, oh1c[c], preferred_element_type=jnp.float32)
        pos0 = jnp.sum(oh0c[c] * (ex0 + c0[c:c + 1, :] + aoff),
                       axis=1, keepdims=True)
        pos1 = jnp.sum(oh1c[c] * (ex1 + c1[c:c + 1, :] + aoff),
                       axis=1, keepdims=True)
        pall_ref[c * BLK:(c + 1) * BLK, :] = pos0.astype(jnp.int32)
        pall_ref[N + c * BLK:N + (c + 1) * BLK, :] = pos1.astype(jnp.int32)


def _route(x_flat, wg):
    outs = (
        jax.ShapeDtypeStruct((N, E), jnp.float32),
        jax.ShapeDtypeStruct((NA, 1), jnp.int32),
        jax.ShapeDtypeStruct((N, 1), jnp.float32),
        jax.ShapeDtypeStruct((N, 1), jnp.float32),
        jax.ShapeDtypeStruct((1, E), jnp.int32),
        jax.ShapeDtypeStruct((1, E), jnp.int32),
    )
    return pl.pallas_call(_route_body, out_shape=outs)(x_flat, wg)


_NW = 32
_TOK_W = N // _NW
_ROWS_W = NA // _NW


def _sc_mesh():
    return plsc.VectorSubcoreMesh(core_axis_name="c", subcore_axis_name="s")


def _sc_scatter(x_flat, pall):
    @functools.partial(
        pl.kernel,
        mesh=_sc_mesh(),
        out_type=jax.ShapeDtypeStruct((P_PAD, C), jnp.float32),
        scratch_types=[
            pltpu.VMEM((_ROWS_W,), jnp.int32),
            pltpu.VMEM((_ROWS_W, C), jnp.float32),
            pltpu.SemaphoreType.DMA,
            pltpu.SemaphoreType.DMA,
        ],
    )
    def k(x_hbm, idx_hbm, out_hbm, idx_v, rows_v, sem_i, sem_x):
        wid = lax.axis_index("s") * 2 + lax.axis_index("c")
        kk = wid % 2
        ww = wid // 2
        ci = pltpu.async_copy(idx_hbm.at[pl.ds(kk * N + ww * _ROWS_W, _ROWS_W)],
                              idx_v, sem_i)
        cx = pltpu.async_copy(x_hbm.at[pl.ds(ww * _ROWS_W, _ROWS_W)],
                              rows_v, sem_x)
        ci.wait()
        cx.wait()
        pltpu.sync_copy(rows_v, out_hbm.at[idx_v])

    return k(x_flat, pall)


def _sc_gather(table, pall):
    @functools.partial(
        pl.kernel,
        mesh=_sc_mesh(),
        out_type=jax.ShapeDtypeStruct((NA, C), jnp.float32),
        scratch_types=[
            pltpu.VMEM((_ROWS_W,), jnp.int32),
            pltpu.VMEM((_ROWS_W, C), jnp.float32),
            pltpu.SemaphoreType.DMA,
        ],
    )
    def k(tab_hbm, idx_hbm, out_hbm, idx_v, rows_v, sem):
        wid = lax.axis_index("s") * 2 + lax.axis_index("c")
        base = wid * _ROWS_W
        pltpu.sync_copy(idx_hbm.at[pl.ds(base, _ROWS_W)], idx_v)
        pltpu.async_copy(tab_hbm.at[idx_v], rows_v, sem).wait()
        pltpu.sync_copy(rows_v, out_hbm.at[pl.ds(base, _ROWS_W)])

    return k(table, pall)


EPS = 4


def _gmm_step(eoff_ref, eblk_ref, xs_ref, w1_ref, w2_ref, out_ref):
    s = pl.program_id(0)

    def _mlp(w1, w2, r0, m):
        xb = xs_ref[pl.ds(r0, m), :]
        h = lax.dot_general(xb, w1, (((1,), (1,)), ((), ())),
                            preferred_element_type=jnp.float32,
                            precision=lax.Precision.DEFAULT)
        h = jnp.square(jnp.maximum(h, 0.0))
        out_ref[pl.ds(r0, m), :] = lax.dot_general(
            h, w2, (((1,), (1,)), ((), ())),
            preferred_element_type=jnp.float32,
            precision=lax.Precision.DEFAULT)

    for i in range(EPS):
        e = s * EPS + i
        off = eoff_ref[e]
        nb = eblk_ref[e]
        w1 = w1_ref[i]
        w2 = w2_ref[i]

        def body(j, carry, w1=w1, w2=w2, off=off):
            _mlp(w1, w2, off * BLK + j * (2 * BLK), 2 * BLK)
            return carry

        lax.fori_loop(0, nb // 2, body, 0)

        @pl.when(nb % 2 == 1)
        def _tail(w1=w1, w2=w2, off=off, nb=nb):
            _mlp(w1, w2, (off + nb - 1) * BLK, BLK)


def _gmm(eoff, eblk, xs, w1, w2):
    grid_spec = pltpu.PrefetchScalarGridSpec(
        num_scalar_prefetch=2,
        grid=(E // EPS,),
        in_specs=[
            pl.BlockSpec((P_PAD, C), lambda s, o, nb: (0, 0)),
            pl.BlockSpec((EPS, FF, C), lambda s, o, nb: (s, 0, 0)),
            pl.BlockSpec((EPS, C, FF), lambda s, o, nb: (s, 0, 0)),
        ],
        out_specs=pl.BlockSpec((P_PAD, C), lambda s, o, nb: (0, 0)),
    )
    return pl.pallas_call(
        _gmm_step,
        grid_spec=grid_spec,
        out_shape=jax.ShapeDtypeStruct((P_PAD, C), jnp.float32),
    )(eoff, eblk, xs, w1, w2)


def _combine_body(g_ref, w0_ref, w1_ref, out_ref):
    out_ref[...] = (g_ref[0:N, :] * w0_ref[...]
                    + g_ref[N:NA, :] * w1_ref[...])


def _combine(g, w0, w1):
    return pl.pallas_call(
        _combine_body,
        out_shape=jax.ShapeDtypeStruct((N, C), jnp.float32),
    )(g, w0, w1)


def kernel(x, Wg, W1, W2):
    bsz, t, c = x.shape
    x_flat = x.reshape(N, C)
    rs, pall, w0, w1, eoff, eblk = _route(x_flat, Wg)
    pf = pall.reshape(-1)
    xs = _sc_scatter(x_flat, pf)
    outs = _gmm(eoff.reshape(-1), eblk.reshape(-1), xs, W1, W2)
    g = _sc_gather(outs, pf)
    out = _combine(g, w0, w1)
    return out.reshape(bsz, t, c), rs
